# Initial kernel scaffold; baseline (speedup 1.0000x reference)
import jax
import jax.numpy as jnp
from jax.experimental import pallas as pl


def kernel(hidden_states, x_pack_kwargs):
    raise NotImplementedError("write your pallas kernel here")


# speedup vs baseline: 1.6465x; 1.6465x over previous
import functools

import jax
import jax.numpy as jnp
from jax.experimental import pallas as pl
from jax.experimental.pallas import tpu as pltpu

L = 32768
D = 1024
N = 16
BL = 512
EPS = 1e-6


def _tc_body(lens_ref, x_ref, mask_ref, rp_ref, sel_ref, cu_ref,
             carry_ref, cnt_ref):
    i = pl.program_id(0)
    nb = pl.num_programs(0)

    x = x_ref[:, :]
    sumsq = jnp.sum(x * x, axis=1, keepdims=True)
    xn = x / (jnp.sqrt(sumsq) + EPS)

    prev = jnp.where(i == 0, xn[0:1, :], carry_ref[0:1, :])
    shifted = jnp.concatenate([prev, xn[:-1, :]], axis=0)
    cos = jnp.sum(xn * shifted, axis=1, keepdims=True)
    p = jnp.clip(0.5 * (1.0 - cos), 0.0, 1.0)

    gpos = i * BL + jax.lax.broadcasted_iota(jnp.int32, (BL, 1), 0)

    cs = jnp.int32(0)
    bmask = jnp.zeros((BL, 1), dtype=jnp.bool_)
    for k in range(N):
        bmask = jnp.logical_or(bmask, gpos == cs)
        cs = cs + lens_ref[0, k]
    p = jnp.where(bmask, 1.0, p)

    mask = p > 0.5
    mask_ref[:, :] = mask.astype(jnp.int8)
    rp_ref[:, :] = jnp.concatenate([1.0 - p, p], axis=1)
    sel_ref[:, :] = jnp.maximum(p, 1.0 - p)

    local = jnp.sum(mask.astype(jnp.int32))
    total = jnp.where(i == 0, local, cnt_ref[0] + local)
    cnt_ref[0] = total

    carry_ref[0:1, :] = xn[BL - 1:BL, :]

    @pl.when(i == nb - 1)
    def _():
        cu_ref[0] = 0
        cu_ref[1] = total


@jax.jit
def kernel(hidden_states, x_pack_kwargs):
    x = hidden_states.reshape(L, D)
    grid = (L // BL,)
    mask8, rp, sel, cu = pl.pallas_call(
        _tc_body,
        grid=grid,
        in_specs=[
            pl.BlockSpec(memory_space=pltpu.SMEM),
            pl.BlockSpec((BL, D), lambda i: (i, 0)),
        ],
        out_specs=[
            pl.BlockSpec((BL, 1), lambda i: (i, 0)),
            pl.BlockSpec((BL, 2), lambda i: (i, 0)),
            pl.BlockSpec((BL, 1), lambda i: (i, 0)),
            pl.BlockSpec(memory_space=pltpu.SMEM),
        ],
        out_shape=[
            jax.ShapeDtypeStruct((L, 1), jnp.int8),
            jax.ShapeDtypeStruct((L, 2), jnp.float32),
            jax.ShapeDtypeStruct((L, 1), jnp.float32),
            jax.ShapeDtypeStruct((2,), jnp.int32),
        ],
        scratch_shapes=[
            pltpu.VMEM((8, D), jnp.float32),
            pltpu.SMEM((1,), jnp.int32),
        ],
    )(x_pack_kwargs, x)

    token_mask = mask8.reshape(1, L).astype(jnp.bool_)
    router_probs = rp.reshape(1, L, 2)
    selected_probs = sel.reshape(1, L, 1)
    return (token_mask, router_probs, selected_probs, cu)
